# baseline (speedup 1.0000x reference)
import functools

import jax
import jax.numpy as jnp
from jax import lax
from jax.experimental import pallas as pl
from jax.experimental.pallas import tpu as pltpu
from jax.experimental.pallas import tpu_sc as plsc

BATCH = 16384
DIM = 128

_NC, _NS = 2, 16
_NW = _NC * _NS
_CH = 128
_NBUF = 4


@functools.lru_cache(maxsize=None)
def _make_sc_gather(nrows):
    bpw = nrows // _NW
    nch = bpw // _CH
    mesh = plsc.VectorSubcoreMesh(core_axis_name="c", subcore_axis_name="s",
                                  num_cores=_NC, num_subcores=_NS)

    @functools.partial(
        pl.kernel,
        mesh=mesh,
        out_type=(
            jax.ShapeDtypeStruct((nrows, DIM), jnp.float32),
            jax.ShapeDtypeStruct((nrows, DIM), jnp.float32),
        ),
        scratch_types=[
            pltpu.VMEM((2 * nch, _CH), jnp.int32),
            pltpu.VMEM((_NBUF * _CH, DIM), jnp.float32),
            pltpu.SemaphoreType.DMA,
            pltpu.SemaphoreType.DMA,
        ],
    )
    def gather_k(su_hbm, ti_hbm, user_hbm, item_hbm, out_u, out_i,
                 idx_v, rows_v, gsem, wsem):
        wid = lax.axis_index("s") * _NC + lax.axis_index("c")
        base = wid * bpw
        pltpu.sync_copy(user_hbm.at[pl.ds(wid * nch, nch)],
                        idx_v.at[pl.ds(0, nch)])
        pltpu.sync_copy(item_hbm.at[pl.ds(wid * nch, nch)],
                        idx_v.at[pl.ds(nch, nch)])
        nchunk = 2 * nch
        chunks = []
        for k in range(nchunk):
            table, out = ((su_hbm, out_u), (ti_hbm, out_i))[k // nch]
            off = base + (k % nch) * _CH
            chunks.append((table, out, off))

        def gather_start(k):
            table, _, _ = chunks[k]
            return pltpu.async_copy(table.at[idx_v.at[k]],
                                    rows_v.at[pl.ds((k % _NBUF) * _CH, _CH)],
                                    gsem)

        writes = [None] * nchunk
        gathers = [None] * nchunk
        gathers[0] = gather_start(0)
        for k in range(nchunk):
            if k + 1 < nchunk:
                if k + 1 >= _NBUF:
                    writes[k + 1 - _NBUF].wait()
                gathers[k + 1] = gather_start(k + 1)
            gathers[k].wait()
            _, out, off = chunks[k]
            writes[k] = pltpu.async_copy(
                rows_v.at[pl.ds((k % _NBUF) * _CH, _CH)],
                out.at[pl.ds(off, _CH)], wsem)
        for k in range(max(0, nchunk - _NBUF), nchunk):
            writes[k].wait()

    return gather_k


_BB = 2048


def _tc_body(u_ref, it_ref, w1, b1, w2, b2, w3, b3, w4, b4, out_ref):
    bf = jnp.bfloat16
    x = u_ref[...].astype(bf)
    h = jnp.maximum(jnp.dot(x, w1[...],
                            preferred_element_type=jnp.float32).astype(bf)
                    + b1[...], 0)
    h = jnp.maximum(jnp.dot(h, w2[...],
                            preferred_element_type=jnp.float32).astype(bf)
                    + b2[...], 0)
    h = jnp.maximum(jnp.dot(h, w3[...],
                            preferred_element_type=jnp.float32).astype(bf)
                    + b3[...], 0)
    f = jnp.dot(h, w4[...],
                preferred_element_type=jnp.float32).astype(bf) + b4[...]
    ones = jnp.full((1, DIM), 1.0 / 3.0, dtype=bf)
    s = jax.lax.dot_general(ones, f * it_ref[...].astype(bf),
                            (((1,), (1,)), ((), ())),
                            preferred_element_type=jnp.float32)
    out_ref[...] = s[0]


def _tc_mlp_score(u, it, w1, b1, w2, b2, w3, b3, w4, b4):
    nrows = u.shape[0]
    grid = nrows // _BB
    full = lambda shape: pl.BlockSpec(shape, lambda i: (0, 0))
    return pl.pallas_call(
        _tc_body,
        grid=(grid,),
        in_specs=[
            pl.BlockSpec((_BB, DIM), lambda i: (i, 0)),
            pl.BlockSpec((_BB, DIM), lambda i: (i, 0)),
            full(w1.shape), full(b1.shape),
            full(w2.shape), full(b2.shape),
            full(w3.shape), full(b3.shape),
            full(w4.shape), full(b4.shape),
        ],
        out_specs=pl.BlockSpec((_BB,), lambda i: (i,)),
        out_shape=jax.ShapeDtypeStruct((nrows,), jnp.float32),
    )(u, it, w1, b1, w2, b2, w3, b3, w4, b4)


_NSPLIT = 1


def kernel(user, item, su_emb, ti_emb, mlp1, mlp2, mlp3):
    mlps = (mlp1, mlp2, mlp3)
    bf = jnp.bfloat16
    w1 = jnp.concatenate([m[0][0] for m in mlps], axis=1).astype(bf)
    b1 = jnp.concatenate([m[0][1] for m in mlps])[None, :].astype(bf)
    w2 = jax.scipy.linalg.block_diag(*[m[1][0] for m in mlps]).astype(bf)
    b2 = jnp.concatenate([m[1][1] for m in mlps])[None, :].astype(bf)
    w3 = jax.scipy.linalg.block_diag(*[m[2][0] for m in mlps]).astype(bf)
    b3 = jnp.concatenate([m[2][1] for m in mlps])[None, :].astype(bf)
    w4 = jnp.concatenate([m[3][0] for m in mlps], axis=0).astype(bf)
    b4 = (mlp1[3][1] + mlp2[3][1] + mlp3[3][1])[None, :].astype(bf)

    user = user.astype(jnp.int32)
    item = item.astype(jnp.int32)
    if _NSPLIT == 1:
        u_rows, it_rows = _make_sc_gather(BATCH)(
            su_emb, ti_emb,
            user.reshape(BATCH // _CH, _CH), item.reshape(BATCH // _CH, _CH))
        return _tc_mlp_score(u_rows, it_rows, w1, b1, w2, b2, w3, b3, w4, b4)
    half = BATCH // _NSPLIT
    gather = _make_sc_gather(half)
    scores = []
    for p in range(_NSPLIT):
        u_rows, it_rows = gather(su_emb, ti_emb,
                                 lax.dynamic_slice_in_dim(user, p * half, half),
                                 lax.dynamic_slice_in_dim(item, p * half, half))
        scores.append(_tc_mlp_score(u_rows, it_rows,
                                    w1, b1, w2, b2, w3, b3, w4, b4))
    return jnp.concatenate(scores)


# speedup vs baseline: 1.0089x; 1.0089x over previous
import functools

import jax
import jax.numpy as jnp
from jax import lax
from jax.experimental import pallas as pl
from jax.experimental.pallas import tpu as pltpu
from jax.experimental.pallas import tpu_sc as plsc

BATCH = 16384
DIM = 128

_NC, _NS = 2, 16
_NW = _NC * _NS
_CH = 128
_NBUF = 4


@functools.lru_cache(maxsize=None)
def _make_sc_gather(nrows):
    bpw = nrows // _NW
    nch = bpw // _CH
    mesh = plsc.VectorSubcoreMesh(core_axis_name="c", subcore_axis_name="s",
                                  num_cores=_NC, num_subcores=_NS)

    @functools.partial(
        pl.kernel,
        mesh=mesh,
        out_type=(
            jax.ShapeDtypeStruct((nrows, DIM), jnp.float32),
            jax.ShapeDtypeStruct((nrows, DIM), jnp.float32),
        ),
        scratch_types=[
            pltpu.VMEM((2 * nch, _CH), jnp.int32),
            pltpu.VMEM((_NBUF * _CH, DIM), jnp.float32),
            pltpu.SemaphoreType.DMA,
            pltpu.SemaphoreType.DMA,
        ],
    )
    def gather_k(su_hbm, ti_hbm, user_hbm, item_hbm, out_u, out_i,
                 idx_v, rows_v, gsem, wsem):
        wid = lax.axis_index("s") * _NC + lax.axis_index("c")
        base = wid * bpw
        pltpu.sync_copy(user_hbm.at[pl.ds(wid * nch, nch)],
                        idx_v.at[pl.ds(0, nch)])
        pltpu.sync_copy(item_hbm.at[pl.ds(wid * nch, nch)],
                        idx_v.at[pl.ds(nch, nch)])
        nchunk = 2 * nch
        chunks = []
        for k in range(nchunk):
            table, out = ((su_hbm, out_u), (ti_hbm, out_i))[k // nch]
            off = base + (k % nch) * _CH
            chunks.append((table, out, off))

        def gather_start(k):
            table, _, _ = chunks[k]
            return pltpu.async_copy(table.at[idx_v.at[k]],
                                    rows_v.at[pl.ds((k % _NBUF) * _CH, _CH)],
                                    gsem)

        writes = [None] * nchunk
        gathers = [None] * nchunk
        gathers[0] = gather_start(0)
        for k in range(nchunk):
            if k + 1 < nchunk:
                if k + 1 >= _NBUF:
                    writes[k + 1 - _NBUF].wait()
                gathers[k + 1] = gather_start(k + 1)
            gathers[k].wait()
            _, out, off = chunks[k]
            writes[k] = pltpu.async_copy(
                rows_v.at[pl.ds((k % _NBUF) * _CH, _CH)],
                out.at[pl.ds(off, _CH)], wsem)
        for k in range(max(0, nchunk - _NBUF), nchunk):
            writes[k].wait()

    return gather_k


_BB = 8192


def _tc_body(u_ref, it_ref, w1, b1, w2, b2, w3, b3, w4, b4, out_ref):
    bf = jnp.bfloat16
    x = u_ref[...].astype(bf)
    h = jnp.maximum(jnp.dot(x, w1[...],
                            preferred_element_type=jnp.float32).astype(bf)
                    + b1[...], 0)
    h = jnp.maximum(jnp.dot(h, w2[...],
                            preferred_element_type=jnp.float32).astype(bf)
                    + b2[...], 0)
    h = jnp.maximum(jnp.dot(h, w3[...],
                            preferred_element_type=jnp.float32).astype(bf)
                    + b3[...], 0)
    f = jnp.dot(h, w4[...],
                preferred_element_type=jnp.float32).astype(bf) + b4[...]
    ones = jnp.full((1, DIM), 1.0 / 3.0, dtype=bf)
    s = jax.lax.dot_general(ones, f * it_ref[...].astype(bf),
                            (((1,), (1,)), ((), ())),
                            preferred_element_type=jnp.float32)
    out_ref[...] = s[0]


def _tc_mlp_score(u, it, w1, b1, w2, b2, w3, b3, w4, b4):
    nrows = u.shape[0]
    grid = nrows // _BB
    full = lambda shape: pl.BlockSpec(shape, lambda i: (0, 0))
    return pl.pallas_call(
        _tc_body,
        grid=(grid,),
        in_specs=[
            pl.BlockSpec((_BB, DIM), lambda i: (i, 0)),
            pl.BlockSpec((_BB, DIM), lambda i: (i, 0)),
            full(w1.shape), full(b1.shape),
            full(w2.shape), full(b2.shape),
            full(w3.shape), full(b3.shape),
            full(w4.shape), full(b4.shape),
        ],
        out_specs=pl.BlockSpec((_BB,), lambda i: (i,)),
        out_shape=jax.ShapeDtypeStruct((nrows,), jnp.float32),
    )(u, it, w1, b1, w2, b2, w3, b3, w4, b4)


_NSPLIT = 1


def kernel(user, item, su_emb, ti_emb, mlp1, mlp2, mlp3):
    mlps = (mlp1, mlp2, mlp3)
    bf = jnp.bfloat16
    w1 = jnp.concatenate([m[0][0] for m in mlps], axis=1).astype(bf)
    b1 = jnp.concatenate([m[0][1] for m in mlps])[None, :].astype(bf)
    w2 = jax.scipy.linalg.block_diag(*[m[1][0] for m in mlps]).astype(bf)
    b2 = jnp.concatenate([m[1][1] for m in mlps])[None, :].astype(bf)
    w3 = jax.scipy.linalg.block_diag(*[m[2][0] for m in mlps]).astype(bf)
    b3 = jnp.concatenate([m[2][1] for m in mlps])[None, :].astype(bf)
    w4 = jnp.concatenate([m[3][0] for m in mlps], axis=0).astype(bf)
    b4 = (mlp1[3][1] + mlp2[3][1] + mlp3[3][1])[None, :].astype(bf)

    user = user.astype(jnp.int32)
    item = item.astype(jnp.int32)
    if _NSPLIT == 1:
        u_rows, it_rows = _make_sc_gather(BATCH)(
            su_emb, ti_emb,
            user.reshape(BATCH // _CH, _CH), item.reshape(BATCH // _CH, _CH))
        return _tc_mlp_score(u_rows, it_rows, w1, b1, w2, b2, w3, b3, w4, b4)
    half = BATCH // _NSPLIT
    gather = _make_sc_gather(half)
    scores = []
    for p in range(_NSPLIT):
        u_rows, it_rows = gather(su_emb, ti_emb,
                                 lax.dynamic_slice_in_dim(user, p * half, half),
                                 lax.dynamic_slice_in_dim(item, p * half, half))
        scores.append(_tc_mlp_score(u_rows, it_rows,
                                    w1, b1, w2, b2, w3, b3, w4, b4))
    return jnp.concatenate(scores)


# speedup vs baseline: 1.0238x; 1.0147x over previous
import functools

import jax
import jax.numpy as jnp
from jax import lax
from jax.experimental import pallas as pl
from jax.experimental.pallas import tpu as pltpu
from jax.experimental.pallas import tpu_sc as plsc

BATCH = 16384
DIM = 128

_NC, _NS = 2, 16
_NW = _NC * _NS
_CH = 128
_NBUF = 6
_DEPTH = 3


@functools.lru_cache(maxsize=None)
def _make_sc_gather(nrows):
    bpw = nrows // _NW
    nch = bpw // _CH
    mesh = plsc.VectorSubcoreMesh(core_axis_name="c", subcore_axis_name="s",
                                  num_cores=_NC, num_subcores=_NS)

    @functools.partial(
        pl.kernel,
        mesh=mesh,
        out_type=(
            jax.ShapeDtypeStruct((nrows, DIM), jnp.float32),
            jax.ShapeDtypeStruct((nrows, DIM), jnp.float32),
        ),
        scratch_types=[
            pltpu.VMEM((2 * nch, _CH), jnp.int32),
            pltpu.VMEM((_NBUF * _CH, DIM), jnp.float32),
            pltpu.SemaphoreType.DMA,
            pltpu.SemaphoreType.DMA,
        ],
    )
    def gather_k(su_hbm, ti_hbm, user_hbm, item_hbm, out_u, out_i,
                 idx_v, rows_v, gsem, wsem):
        wid = lax.axis_index("s") * _NC + lax.axis_index("c")
        base = wid * bpw
        pltpu.sync_copy(user_hbm.at[pl.ds(wid * nch, nch)],
                        idx_v.at[pl.ds(0, nch)])
        pltpu.sync_copy(item_hbm.at[pl.ds(wid * nch, nch)],
                        idx_v.at[pl.ds(nch, nch)])
        nchunk = 2 * nch
        chunks = []
        for k in range(nchunk):
            table, out = ((su_hbm, out_u), (ti_hbm, out_i))[k // nch]
            off = base + (k % nch) * _CH
            chunks.append((table, out, off))

        def gather_start(k):
            table, _, _ = chunks[k]
            return pltpu.async_copy(table.at[idx_v.at[k]],
                                    rows_v.at[pl.ds((k % _NBUF) * _CH, _CH)],
                                    gsem)

        depth = min(_DEPTH, nchunk)
        writes = [None] * nchunk
        gathers = [None] * nchunk
        for k in range(depth):
            gathers[k] = gather_start(k)
        for k in range(nchunk):
            nxt = k + depth
            if nxt < nchunk:
                if nxt >= _NBUF:
                    writes[nxt - _NBUF].wait()
                gathers[nxt] = gather_start(nxt)
            gathers[k].wait()
            _, out, off = chunks[k]
            writes[k] = pltpu.async_copy(
                rows_v.at[pl.ds((k % _NBUF) * _CH, _CH)],
                out.at[pl.ds(off, _CH)], wsem)
        for k in range(max(0, nchunk - _NBUF), nchunk):
            writes[k].wait()

    return gather_k


_BB = 4096


def _tc_body(u_ref, it_ref, w1, b1, w2, b2, w3, b3, w4, b4, out_ref):
    bf = jnp.bfloat16
    x = u_ref[...].astype(bf)
    h = jnp.maximum(jnp.dot(x, w1[...],
                            preferred_element_type=jnp.float32).astype(bf)
                    + b1[...], 0)
    h = jnp.maximum(jnp.dot(h, w2[...],
                            preferred_element_type=jnp.float32).astype(bf)
                    + b2[...], 0)
    h = jnp.maximum(jnp.dot(h, w3[...],
                            preferred_element_type=jnp.float32).astype(bf)
                    + b3[...], 0)
    f = jnp.dot(h, w4[...],
                preferred_element_type=jnp.float32).astype(bf) + b4[...]
    ones = jnp.full((1, DIM), 1.0 / 3.0, dtype=bf)
    s = jax.lax.dot_general(ones, f * it_ref[...].astype(bf),
                            (((1,), (1,)), ((), ())),
                            preferred_element_type=jnp.float32)
    out_ref[...] = s[0]


def _tc_mlp_score(u, it, w1, b1, w2, b2, w3, b3, w4, b4):
    nrows = u.shape[0]
    grid = nrows // _BB
    full = lambda shape: pl.BlockSpec(shape, lambda i: (0, 0))
    return pl.pallas_call(
        _tc_body,
        grid=(grid,),
        in_specs=[
            pl.BlockSpec((_BB, DIM), lambda i: (i, 0)),
            pl.BlockSpec((_BB, DIM), lambda i: (i, 0)),
            full(w1.shape), full(b1.shape),
            full(w2.shape), full(b2.shape),
            full(w3.shape), full(b3.shape),
            full(w4.shape), full(b4.shape),
        ],
        out_specs=pl.BlockSpec((_BB,), lambda i: (i,)),
        out_shape=jax.ShapeDtypeStruct((nrows,), jnp.float32),
    )(u, it, w1, b1, w2, b2, w3, b3, w4, b4)


_NSPLIT = 1


def kernel(user, item, su_emb, ti_emb, mlp1, mlp2, mlp3):
    mlps = (mlp1, mlp2, mlp3)
    bf = jnp.bfloat16
    w1 = jnp.concatenate([m[0][0] for m in mlps], axis=1).astype(bf)
    b1 = jnp.concatenate([m[0][1] for m in mlps])[None, :].astype(bf)
    w2 = jax.scipy.linalg.block_diag(*[m[1][0] for m in mlps]).astype(bf)
    b2 = jnp.concatenate([m[1][1] for m in mlps])[None, :].astype(bf)
    w3 = jax.scipy.linalg.block_diag(*[m[2][0] for m in mlps]).astype(bf)
    b3 = jnp.concatenate([m[2][1] for m in mlps])[None, :].astype(bf)
    w4 = jnp.concatenate([m[3][0] for m in mlps], axis=0).astype(bf)
    b4 = (mlp1[3][1] + mlp2[3][1] + mlp3[3][1])[None, :].astype(bf)

    user = user.astype(jnp.int32)
    item = item.astype(jnp.int32)
    if _NSPLIT == 1:
        u_rows, it_rows = _make_sc_gather(BATCH)(
            su_emb, ti_emb,
            user.reshape(BATCH // _CH, _CH), item.reshape(BATCH // _CH, _CH))
        return _tc_mlp_score(u_rows, it_rows, w1, b1, w2, b2, w3, b3, w4, b4)
    half = BATCH // _NSPLIT
    gather = _make_sc_gather(half)
    scores = []
    for p in range(_NSPLIT):
        u_rows, it_rows = gather(su_emb, ti_emb,
                                 lax.dynamic_slice_in_dim(user, p * half, half),
                                 lax.dynamic_slice_in_dim(item, p * half, half))
        scores.append(_tc_mlp_score(u_rows, it_rows,
                                    w1, b1, w2, b2, w3, b3, w4, b4))
    return jnp.concatenate(scores)


# speedup vs baseline: 1.0275x; 1.0036x over previous
import functools

import jax
import jax.numpy as jnp
from jax import lax
from jax.experimental import pallas as pl
from jax.experimental.pallas import tpu as pltpu
from jax.experimental.pallas import tpu_sc as plsc

BATCH = 16384
DIM = 128

_NC, _NS = 2, 16
_NW = _NC * _NS
_CH = 128
_NBUF = 6
_DEPTH = 3


@functools.lru_cache(maxsize=None)
def _make_sc_gather(nrows):
    bpw = nrows // _NW
    nch = bpw // _CH
    mesh = plsc.VectorSubcoreMesh(core_axis_name="c", subcore_axis_name="s",
                                  num_cores=_NC, num_subcores=_NS)

    @functools.partial(
        pl.kernel,
        mesh=mesh,
        out_type=(
            jax.ShapeDtypeStruct((nrows, DIM), jnp.float32),
            jax.ShapeDtypeStruct((nrows, DIM), jnp.float32),
        ),
        scratch_types=[
            pltpu.VMEM((2 * nch, _CH), jnp.int32),
            pltpu.VMEM((_NBUF * _CH, DIM), jnp.float32),
            pltpu.SemaphoreType.DMA,
            pltpu.SemaphoreType.DMA,
        ],
    )
    def gather_k(su_hbm, ti_hbm, user_hbm, item_hbm, out_u, out_i,
                 idx_v, rows_v, gsem, wsem):
        wid = lax.axis_index("s") * _NC + lax.axis_index("c")
        base = wid * bpw
        pltpu.sync_copy(user_hbm.at[pl.ds(wid * nch, nch)],
                        idx_v.at[pl.ds(0, nch)])
        pltpu.sync_copy(item_hbm.at[pl.ds(wid * nch, nch)],
                        idx_v.at[pl.ds(nch, nch)])
        nchunk = 2 * nch
        chunks = []
        for k in range(nchunk):
            table, out = ((su_hbm, out_u), (ti_hbm, out_i))[k // nch]
            off = base + (k % nch) * _CH
            chunks.append((table, out, off))

        def gather_start(k):
            table, _, _ = chunks[k]
            return pltpu.async_copy(table.at[idx_v.at[k]],
                                    rows_v.at[pl.ds((k % _NBUF) * _CH, _CH)],
                                    gsem)

        depth = min(_DEPTH, nchunk)
        writes = [None] * nchunk
        gathers = [None] * nchunk
        for k in range(depth):
            gathers[k] = gather_start(k)
        for k in range(nchunk):
            nxt = k + depth
            if nxt < nchunk:
                if nxt >= _NBUF:
                    writes[nxt - _NBUF].wait()
                gathers[nxt] = gather_start(nxt)
            gathers[k].wait()
            _, out, off = chunks[k]
            writes[k] = pltpu.async_copy(
                rows_v.at[pl.ds((k % _NBUF) * _CH, _CH)],
                out.at[pl.ds(off, _CH)], wsem)
        for k in range(max(0, nchunk - _NBUF), nchunk):
            writes[k].wait()

    return gather_k


_BB = 4096


def _tc_body(u_ref, it_ref, w1, b1, w2, b2, w3, b3, w4, b4, out_ref):
    bf = jnp.bfloat16
    x = u_ref[...].astype(bf)
    h = jnp.maximum(jnp.dot(x, w1[...],
                            preferred_element_type=jnp.float32).astype(bf)
                    + b1[...], 0)
    h = jnp.maximum(jnp.dot(h, w2[...],
                            preferred_element_type=jnp.float32).astype(bf)
                    + b2[...], 0)
    h = jnp.maximum(jnp.dot(h, w3[...],
                            preferred_element_type=jnp.float32).astype(bf)
                    + b3[...], 0)
    f = jnp.dot(h, w4[...],
                preferred_element_type=jnp.float32).astype(bf) + b4[...]
    ones = jnp.full((1, DIM), 1.0 / 3.0, dtype=bf)
    s = jax.lax.dot_general(ones, f * it_ref[...].astype(bf),
                            (((1,), (1,)), ((), ())),
                            preferred_element_type=jnp.float32)
    out_ref[...] = s[0]


def _tc_mlp_score(u, it, w1, b1, w2, b2, w3, b3, w4, b4):
    nrows = u.shape[0]
    grid = nrows // _BB
    full = lambda shape: pl.BlockSpec(shape, lambda i: (0, 0))
    return pl.pallas_call(
        _tc_body,
        grid=(grid,),
        in_specs=[
            pl.BlockSpec((_BB, DIM), lambda i: (i, 0)),
            pl.BlockSpec((_BB, DIM), lambda i: (i, 0)),
            full(w1.shape), full(b1.shape),
            full(w2.shape), full(b2.shape),
            full(w3.shape), full(b3.shape),
            full(w4.shape), full(b4.shape),
        ],
        out_specs=pl.BlockSpec((_BB,), lambda i: (i,)),
        out_shape=jax.ShapeDtypeStruct((nrows,), jnp.float32),
    )(u, it, w1, b1, w2, b2, w3, b3, w4, b4)


def kernel(user, item, su_emb, ti_emb, mlp1, mlp2, mlp3):
    mlps = (mlp1, mlp2, mlp3)
    bf = jnp.bfloat16
    w1 = jnp.concatenate([m[0][0] for m in mlps], axis=1).astype(bf)
    b1 = jnp.concatenate([m[0][1] for m in mlps])[None, :].astype(bf)
    w2 = jax.scipy.linalg.block_diag(*[m[1][0] for m in mlps]).astype(bf)
    b2 = jnp.concatenate([m[1][1] for m in mlps])[None, :].astype(bf)
    w3 = jax.scipy.linalg.block_diag(*[m[2][0] for m in mlps]).astype(bf)
    b3 = jnp.concatenate([m[2][1] for m in mlps])[None, :].astype(bf)
    w4 = jnp.concatenate([m[3][0] for m in mlps], axis=0).astype(bf)
    b4 = (mlp1[3][1] + mlp2[3][1] + mlp3[3][1])[None, :].astype(bf)

    user = user.astype(jnp.int32)
    item = item.astype(jnp.int32)
    u_rows, it_rows = _make_sc_gather(BATCH)(
        su_emb, ti_emb,
        user.reshape(BATCH // _CH, _CH), item.reshape(BATCH // _CH, _CH))
    return _tc_mlp_score(u_rows, it_rows, w1, b1, w2, b2, w3, b3, w4, b4)
